# Initial kernel scaffold; baseline (speedup 1.0000x reference)
import jax
import jax.numpy as jnp
from jax.experimental import pallas as pl


def kernel(x, edge_index, edge_attr, batch, W_gcn, b_gcn, We1, be1, We2, be2, We3, be3, Wb1, bb1, Wout, bout):
    raise NotImplementedError("write your pallas kernel here")


# speedup vs baseline: 11.4318x; 11.4318x over previous
import functools

import jax
import jax.numpy as jnp
from jax import lax
from jax.experimental import pallas as pl
from jax.experimental.pallas import tpu as pltpu
from jax.experimental.pallas import tpu_sc as plsc

N = 10000
E = 320000
F_IN = 128
NHID = 64
G = 128

NC = 2
NS = 16
NW = NC * NS
EPW = E // NW
CK = 80
NCHUNK = EPW // CK
N_PAD = 10240
RPW = N_PAD // NS


def _f32(*shape):
    return jax.ShapeDtypeStruct(shape, jnp.float32)


_EB = 4000


def _bf(a):
    return a.astype(jnp.bfloat16)


def _dot_t(a, b):
    return lax.dot_general(_bf(a), _bf(b), (((1,), (1,)), ((), ())),
                           preferred_element_type=jnp.float32)


def _edge_mlp_body(ea, We1, be1, We2, be2, We3, be3, out):
    h = jnp.maximum(_dot_t(ea[...], We1[...]) + be1[...], 0.0)
    h = jnp.maximum(_dot_t(h, We2[...]) + be2[...], 0.0)
    h = jnp.sum(_bf(h).astype(jnp.float32) * _bf(We3[...]).astype(jnp.float32),
                axis=1, keepdims=True)
    out[...] = jnp.maximum(h + be3[0], 0.0)


def _edge_mlp(edge_attr, We1, be1, We2, be2, We3, be3):
    grid = E // _EB
    full = lambda a: pl.BlockSpec(a.shape, lambda i: (0,) * a.ndim)
    return pl.pallas_call(
        _edge_mlp_body,
        grid=(grid,),
        in_specs=[
            pl.BlockSpec((_EB, 5), lambda i: (i, 0)),
            full(We1), full(be1), full(We2), full(be2), full(We3),
            pl.BlockSpec(memory_space=pltpu.SMEM),
        ],
        out_specs=pl.BlockSpec((_EB, 1), lambda i: (i, 0)),
        out_shape=_f32(E, 1),
    )(edge_attr, We1, be1, We2, be2, We3, be3)


_SC_MESH = plsc.VectorSubcoreMesh(core_axis_name="c", subcore_axis_name="s")
_SC_PARAMS = pltpu.CompilerParams(use_tc_tiling_on_sc=False)


@functools.partial(
    pl.kernel,
    out_type=_f32(NC, N_PAD),
    mesh=_SC_MESH,
    compiler_params=_SC_PARAMS,
    scratch_types=[
        pltpu.VMEM((NCHUNK, CK), jnp.int32),
        pltpu.VMEM((NCHUNK, CK), jnp.float32),
        pltpu.VMEM((RPW,), jnp.float32),
        pltpu.VMEM_SHARED((N_PAD,), jnp.float32),
    ],
)
def _degree(col_hbm, ew_hbm, out_hbm, col_v, ew_v, zb, deg_sh):
    c = lax.axis_index("c")
    s = lax.axis_index("s")
    wid = c * NS + s

    pltpu.sync_copy(col_hbm.at[wid], col_v)
    pltpu.sync_copy(ew_hbm.at[wid], ew_v)

    zero16 = jnp.zeros((16,), jnp.float32)

    @pl.loop(0, RPW // 16)
    def _(i):
        zb[pl.ds(pl.multiple_of(i * 16, 16), 16)] = zero16

    pltpu.sync_copy(zb, deg_sh.at[pl.ds(pl.multiple_of(s * RPW, RPW), RPW)])
    plsc.subcore_barrier()

    @pl.loop(0, NCHUNK)
    def _(j):
        pltpu.sync_copy(ew_v.at[j], deg_sh.at[col_v.at[j]], add=True)

    plsc.subcore_barrier()
    pltpu.sync_copy(deg_sh.at[pl.ds(pl.multiple_of(s * RPW, RPW), RPW)],
                    out_hbm.at[c, pl.ds(pl.multiple_of(s * RPW, RPW), RPW)])


_NB = 1000


def _scale_body(x, Wg, da, db, y, dinv):
    xl = _dot_t(x[...], Wg[...])
    d = da[...] + db[...] + 1.0
    di = lax.rsqrt(d)
    dinv[...] = di
    y[...] = di * xl


def _scale_xw(x, W_gcn, da, db):
    grid = N // _NB
    return pl.pallas_call(
        _scale_body,
        grid=(grid,),
        in_specs=[
            pl.BlockSpec((_NB, F_IN), lambda i: (i, 0)),
            pl.BlockSpec(W_gcn.shape, lambda i: (0, 0)),
            pl.BlockSpec((_NB, 1), lambda i: (i, 0)),
            pl.BlockSpec((_NB, 1), lambda i: (i, 0)),
        ],
        out_specs=[
            pl.BlockSpec((_NB, NHID), lambda i: (i, 0)),
            pl.BlockSpec((_NB, 1), lambda i: (i, 0)),
        ],
        out_shape=[_f32(N, NHID), _f32(N, 1)],
    )(x, W_gcn, da, db)


_ZR = 64


@functools.partial(
    pl.kernel,
    out_type=_f32(NC, N_PAD, NHID),
    mesh=_SC_MESH,
    compiler_params=_SC_PARAMS,
    scratch_types=[
        pltpu.VMEM((NCHUNK, CK), jnp.int32),
        pltpu.VMEM((NCHUNK, CK), jnp.int32),
        pltpu.VMEM((NCHUNK, CK), jnp.float32),
        pltpu.VMEM((CK, NHID), jnp.float32),
        pltpu.VMEM((_ZR, NHID), jnp.float32),
        pltpu.VMEM_SHARED((N_PAD, NHID), jnp.float32),
        pltpu.SemaphoreType.DMA,
    ],
)
def _messages(row_hbm, col_hbm, ew_hbm, y_hbm, out_hbm,
              row_v, col_v, ew_v, rows, zbuf, S, sem):
    c = lax.axis_index("c")
    s = lax.axis_index("s")
    wid = c * NS + s

    pltpu.sync_copy(row_hbm.at[wid], row_v)
    pltpu.sync_copy(col_hbm.at[wid], col_v)
    pltpu.sync_copy(ew_hbm.at[wid], ew_v)

    zero16 = jnp.zeros((16,), jnp.float32)

    @pl.loop(0, _ZR)
    def _(r):
        for q in range(NHID // 16):
            zbuf[r, pl.ds(q * 16, 16)] = zero16

    @pl.loop(0, RPW // _ZR)
    def _(k):
        pltpu.sync_copy(zbuf, S.at[pl.ds(s * RPW + k * _ZR, _ZR)])

    plsc.subcore_barrier()

    @pl.loop(0, NCHUNK)
    def _(j):
        pltpu.async_copy(y_hbm.at[row_v.at[j]], rows, sem).wait()

        @pl.loop(0, CK // 16)
        def _(g):
            ewg = ew_v[j, pl.ds(g * 16, 16)]
            for ee in range(16):
                sv = jnp.zeros((16,), jnp.float32) + ewg[ee]
                e = g * 16 + ee
                for q in range(NHID // 16):
                    rows[e, pl.ds(q * 16, 16)] = rows[e, pl.ds(q * 16, 16)] * sv

        pltpu.sync_copy(rows, S.at[col_v.at[j]], add=True)

    plsc.subcore_barrier()
    pltpu.sync_copy(S.at[pl.ds(s * RPW, RPW)], out_hbm.at[c, pl.ds(s * RPW, RPW)])


def _head_body(s0, s1, y, dinv, batch, bg, Wb1, bb1, Wout, bout, out, acc):
    i = pl.program_id(0)

    @pl.when(i == 0)
    def _():
        acc[...] = jnp.zeros_like(acc)

    o = dinv[...] * (s0[...] + s1[...] + y[...]) + bg[...]
    x1 = jnp.maximum(o, 0.0)
    gids = lax.broadcasted_iota(jnp.int32, (_NB, G), 1)
    oh = (batch[...] == gids).astype(jnp.float32)
    acc[...] += lax.dot_general(oh, x1, (((0,), (0,)), ((), ())),
                                preferred_element_type=jnp.float32,
                                precision=lax.Precision.HIGHEST)

    @pl.when(i == pl.num_programs(0) - 1)
    def _():
        hb = jnp.maximum(_dot_t(acc[...], Wb1[...]) + bb1[...], 0.0)
        lg = jnp.sum(_bf(hb).astype(jnp.float32)
                     * _bf(Wout[...]).astype(jnp.float32),
                     axis=1, keepdims=True)
        out[...] = lg + bout[0]


def _head(s0, s1, y, dinv, batch, bg, Wb1, bb1, Wout, bout):
    grid = N // _NB
    full = lambda a: pl.BlockSpec(a.shape, lambda i: (0,) * a.ndim)
    blk = lambda w: pl.BlockSpec((_NB, w), lambda i: (i, 0))
    return pl.pallas_call(
        _head_body,
        grid=(grid,),
        in_specs=[blk(NHID), blk(NHID), blk(NHID), blk(1), blk(1),
                  full(bg), full(Wb1), full(bb1), full(Wout),
                  pl.BlockSpec(memory_space=pltpu.SMEM)],
        out_specs=pl.BlockSpec((G, 1), lambda i: (0, 0)),
        out_shape=_f32(G, 1),
        scratch_shapes=[pltpu.VMEM((G, NHID), jnp.float32)],
    )(s0, s1, y, dinv, batch, bg, Wb1, bb1, Wout, bout)


def kernel(x, edge_index, edge_attr, batch, W_gcn, b_gcn, We1, be1, We2, be2,
           We3, be3, Wb1, bb1, Wout, bout):
    row = edge_index[0]
    col = edge_index[1]

    ew2 = _edge_mlp(edge_attr, We1, be1.reshape(1, NHID), We2,
                    be2.reshape(1, NHID), We3, be3)
    ew = ew2.reshape(E)

    deg2 = _degree(col.reshape(NW, NCHUNK, CK), ew.reshape(NW, NCHUNK, CK))
    da = deg2[0, :N].reshape(N, 1)
    db = deg2[1, :N].reshape(N, 1)

    y, dinv = _scale_xw(x, W_gcn, da, db)

    s2 = _messages(row.reshape(NW, NCHUNK, CK), col.reshape(NW, NCHUNK, CK),
                   ew.reshape(NW, NCHUNK, CK), y)
    s0 = s2[0, :N]
    s1 = s2[1, :N]

    return _head(s0, s1, y, dinv, batch.reshape(N, 1), b_gcn.reshape(1, NHID),
                 Wb1, bb1.reshape(1, NHID), Wout, bout)


# speedup vs baseline: 12.1921x; 1.0665x over previous
import functools

import jax
import jax.numpy as jnp
from jax import lax
from jax.experimental import pallas as pl
from jax.experimental.pallas import tpu as pltpu
from jax.experimental.pallas import tpu_sc as plsc

N = 10000
E = 320000
F_IN = 128
NHID = 64
G = 128

NC = 2
NS = 16
NW = NC * NS
EPW = E // NW
CK = 80
NCHUNK = 128
EPW_P = NCHUNK * CK
NBUF = 8
PF = NBUF - 1
N_PAD = 10240
RPW = N_PAD // NS


def _f32(*shape):
    return jax.ShapeDtypeStruct(shape, jnp.float32)


_EB = 4000


def _bf(a):
    return a.astype(jnp.bfloat16)


def _dot_t(a, b):
    return lax.dot_general(_bf(a), _bf(b), (((1,), (1,)), ((), ())),
                           preferred_element_type=jnp.float32)


def _edge_mlp_body(ea, We1, be1, We2, be2, We3, be3, out):
    h = jnp.maximum(_dot_t(ea[...], We1[...]) + be1[...], 0.0)
    h = jnp.maximum(_dot_t(h, We2[...]) + be2[...], 0.0)
    h = jnp.sum(_bf(h).astype(jnp.float32) * _bf(We3[...]).astype(jnp.float32),
                axis=1, keepdims=True)
    out[...] = jnp.maximum(h + be3[0], 0.0)


def _edge_mlp(edge_attr, We1, be1, We2, be2, We3, be3):
    grid = E // _EB
    full = lambda a: pl.BlockSpec(a.shape, lambda i: (0,) * a.ndim)
    return pl.pallas_call(
        _edge_mlp_body,
        grid=(grid,),
        in_specs=[
            pl.BlockSpec((_EB, 5), lambda i: (i, 0)),
            full(We1), full(be1), full(We2), full(be2), full(We3),
            pl.BlockSpec(memory_space=pltpu.SMEM),
        ],
        out_specs=pl.BlockSpec((_EB, 1), lambda i: (i, 0)),
        out_shape=_f32(E, 1),
    )(edge_attr, We1, be1, We2, be2, We3, be3)


_SC_MESH = plsc.VectorSubcoreMesh(core_axis_name="c", subcore_axis_name="s")
_SC_PARAMS = pltpu.CompilerParams(use_tc_tiling_on_sc=False)


@functools.partial(
    pl.kernel,
    out_type=_f32(NC, N_PAD),
    mesh=_SC_MESH,
    compiler_params=_SC_PARAMS,
    scratch_types=[
        pltpu.VMEM((NCHUNK, CK), jnp.int32),
        pltpu.VMEM((NCHUNK, CK), jnp.float32),
        pltpu.VMEM((RPW,), jnp.float32),
        pltpu.VMEM_SHARED((N_PAD,), jnp.float32),
    ],
)
def _degree(col_hbm, ew_hbm, out_hbm, col_v, ew_v, zb, deg_sh):
    c = lax.axis_index("c")
    s = lax.axis_index("s")
    wid = c * NS + s

    pltpu.sync_copy(col_hbm.at[wid], col_v)
    pltpu.sync_copy(ew_hbm.at[wid], ew_v)

    zero16 = jnp.zeros((16,), jnp.float32)

    @pl.loop(0, RPW // 16)
    def _(i):
        zb[pl.ds(pl.multiple_of(i * 16, 16), 16)] = zero16

    pltpu.sync_copy(zb, deg_sh.at[pl.ds(pl.multiple_of(s * RPW, RPW), RPW)])
    plsc.subcore_barrier()

    @pl.loop(0, NCHUNK)
    def _(j):
        pltpu.sync_copy(ew_v.at[j], deg_sh.at[col_v.at[j]], add=True)

    plsc.subcore_barrier()
    pltpu.sync_copy(deg_sh.at[pl.ds(pl.multiple_of(s * RPW, RPW), RPW)],
                    out_hbm.at[c, pl.ds(pl.multiple_of(s * RPW, RPW), RPW)])


_NB = 1000


def _scale_body(x, Wg, da, db, y, dinv):
    xl = _dot_t(x[...], Wg[...])
    d = da[...] + db[...] + 1.0
    di = lax.rsqrt(d)
    dinv[...] = di
    y[...] = di * xl


def _scale_xw(x, W_gcn, da, db):
    grid = N // _NB
    return pl.pallas_call(
        _scale_body,
        grid=(grid,),
        in_specs=[
            pl.BlockSpec((_NB, F_IN), lambda i: (i, 0)),
            pl.BlockSpec(W_gcn.shape, lambda i: (0, 0)),
            pl.BlockSpec((_NB, 1), lambda i: (i, 0)),
            pl.BlockSpec((_NB, 1), lambda i: (i, 0)),
        ],
        out_specs=[
            pl.BlockSpec((_NB, NHID), lambda i: (i, 0)),
            pl.BlockSpec((_NB, 1), lambda i: (i, 0)),
        ],
        out_shape=[_f32(N, NHID), _f32(N, 1)],
    )(x, W_gcn, da, db)


_ZR = 64


@functools.partial(
    pl.kernel,
    out_type=_f32(NC, N_PAD, NHID),
    mesh=_SC_MESH,
    compiler_params=_SC_PARAMS,
    scratch_types=[
        pltpu.VMEM((NCHUNK, CK), jnp.int32),
        pltpu.VMEM((NCHUNK, CK), jnp.int32),
        pltpu.VMEM((NCHUNK, CK), jnp.float32),
        pltpu.VMEM((NBUF, CK, NHID), jnp.float32),
        pltpu.VMEM((_ZR, NHID), jnp.float32),
        pltpu.VMEM_SHARED((N_PAD, NHID), jnp.float32),
        pltpu.SemaphoreType.DMA((NBUF,)),
        pltpu.SemaphoreType.DMA((NBUF,)),
    ],
)
def _messages(row_hbm, col_hbm, ew_hbm, y_hbm, out_hbm,
              row_v, col_v, ew_v, rows, zbuf, S, gsem, ssem):
    c = lax.axis_index("c")
    s = lax.axis_index("s")
    wid = c * NS + s

    pltpu.sync_copy(row_hbm.at[wid], row_v)
    pltpu.sync_copy(col_hbm.at[wid], col_v)
    pltpu.sync_copy(ew_hbm.at[wid], ew_v)

    zero16 = jnp.zeros((16,), jnp.float32)

    @pl.loop(0, _ZR)
    def _(r):
        for q in range(NHID // 16):
            zbuf[r, pl.ds(q * 16, 16)] = zero16

    @pl.loop(0, RPW // _ZR)
    def _(k):
        pltpu.sync_copy(zbuf, S.at[pl.ds(s * RPW + k * _ZR, _ZR)])

    plsc.subcore_barrier()

    def start_gather(j, b):
        pltpu.async_copy(y_hbm.at[row_v.at[j]], rows.at[b], gsem.at[b])

    for b in range(PF):
        start_gather(b, b)

    @pl.loop(0, NCHUNK, step=NBUF)
    def _(j0):
        for b in range(NBUF):
            j = j0 + b
            pltpu.make_async_copy(y_hbm.at[row_v.at[j]], rows.at[b],
                                  gsem.at[b]).wait()

            @pl.loop(0, CK // 16)
            def _(g):
                ewg = ew_v[j, pl.ds(g * 16, 16)]
                for ee in range(16):
                    sv = jnp.zeros((16,), jnp.float32) + ewg[ee]
                    e = g * 16 + ee
                    for q in range(NHID // 16):
                        rows[b, e, pl.ds(q * 16, 16)] = (
                            rows[b, e, pl.ds(q * 16, 16)] * sv)

            pltpu.async_copy(rows.at[b], S.at[col_v.at[j]], ssem.at[b],
                             add=True)

            bn = (b + PF) % NBUF

            @pl.when(j >= 1)
            def _():
                pltpu.make_async_copy(rows.at[bn], S.at[col_v.at[j - 1]],
                                      ssem.at[bn]).wait()

            @pl.when(j + PF < NCHUNK)
            def _():
                start_gather(j + PF, bn)

    bl = (NCHUNK - 1) % NBUF
    pltpu.make_async_copy(rows.at[bl], S.at[col_v.at[NCHUNK - 1]],
                          ssem.at[bl]).wait()

    plsc.subcore_barrier()
    pltpu.sync_copy(S.at[pl.ds(s * RPW, RPW)], out_hbm.at[c, pl.ds(s * RPW, RPW)])


def _head_body(s0, s1, y, dinv, batch, bg, Wb1, bb1, Wout, bout, out, acc):
    i = pl.program_id(0)

    @pl.when(i == 0)
    def _():
        acc[...] = jnp.zeros_like(acc)

    o = dinv[...] * (s0[...] + s1[...] + y[...]) + bg[...]
    x1 = jnp.maximum(o, 0.0)
    gids = lax.broadcasted_iota(jnp.int32, (_NB, G), 1)
    oh = (batch[...] == gids).astype(jnp.float32)
    acc[...] += lax.dot_general(oh, x1, (((0,), (0,)), ((), ())),
                                preferred_element_type=jnp.float32,
                                precision=lax.Precision.HIGHEST)

    @pl.when(i == pl.num_programs(0) - 1)
    def _():
        hb = jnp.maximum(_dot_t(acc[...], Wb1[...]) + bb1[...], 0.0)
        lg = jnp.sum(_bf(hb).astype(jnp.float32)
                     * _bf(Wout[...]).astype(jnp.float32),
                     axis=1, keepdims=True)
        out[...] = lg + bout[0]


def _head(s0, s1, y, dinv, batch, bg, Wb1, bb1, Wout, bout):
    grid = N // _NB
    full = lambda a: pl.BlockSpec(a.shape, lambda i: (0,) * a.ndim)
    blk = lambda w: pl.BlockSpec((_NB, w), lambda i: (i, 0))
    return pl.pallas_call(
        _head_body,
        grid=(grid,),
        in_specs=[blk(NHID), blk(NHID), blk(NHID), blk(1), blk(1),
                  full(bg), full(Wb1), full(bb1), full(Wout),
                  pl.BlockSpec(memory_space=pltpu.SMEM)],
        out_specs=pl.BlockSpec((G, 1), lambda i: (0, 0)),
        out_shape=_f32(G, 1),
        scratch_shapes=[pltpu.VMEM((G, NHID), jnp.float32)],
    )(s0, s1, y, dinv, batch, bg, Wb1, bb1, Wout, bout)


def kernel(x, edge_index, edge_attr, batch, W_gcn, b_gcn, We1, be1, We2, be2,
           We3, be3, Wb1, bb1, Wout, bout):
    row = edge_index[0]
    col = edge_index[1]

    ew2 = _edge_mlp(edge_attr, We1, be1.reshape(1, NHID), We2,
                    be2.reshape(1, NHID), We3, be3)
    ew = ew2.reshape(E)

    pad = ((0, 0), (0, EPW_P - EPW))
    row_c = jnp.pad(row.reshape(NW, EPW), pad).reshape(NW, NCHUNK, CK)
    col_c = jnp.pad(col.reshape(NW, EPW), pad).reshape(NW, NCHUNK, CK)
    ew_c = jnp.pad(ew.reshape(NW, EPW), pad).reshape(NW, NCHUNK, CK)

    deg2 = _degree(col_c, ew_c)
    da = deg2[0, :N].reshape(N, 1)
    db = deg2[1, :N].reshape(N, 1)

    y, dinv = _scale_xw(x, W_gcn, da, db)

    s2 = _messages(row_c, col_c, ew_c, y)
    s0 = s2[0, :N]
    s1 = s2[1, :N]

    return _head(s0, s1, y, dinv, batch.reshape(N, 1), b_gcn.reshape(1, NHID),
                 Wb1, bb1.reshape(1, NHID), Wout, bout)


# speedup vs baseline: 13.1507x; 1.0786x over previous
import functools

import jax
import jax.numpy as jnp
from jax import lax
from jax.experimental import pallas as pl
from jax.experimental.pallas import tpu as pltpu
from jax.experimental.pallas import tpu_sc as plsc

N = 10000
E = 320000
F_IN = 128
NHID = 64
G = 128

NC = 2
NS = 16
NW = NC * NS
EPW = E // NW
CK = 80
NCHUNK = 128
EPW_P = NCHUNK * CK
NBUF = 4
PF = NBUF - 1
N_PAD = 10240
RPW = N_PAD // NS


def _f32(*shape):
    return jax.ShapeDtypeStruct(shape, jnp.float32)


_EB = 4000


def _bf(a):
    return a.astype(jnp.bfloat16)


def _dot_t(a, b):
    return lax.dot_general(_bf(a), _bf(b), (((1,), (1,)), ((), ())),
                           preferred_element_type=jnp.float32)


def _edge_mlp_body(ea, We1, be1, We2, be2, We3, be3, out):
    h = jnp.maximum(_dot_t(ea[...], We1[...]) + be1[...], 0.0)
    h = jnp.maximum(_dot_t(h, We2[...]) + be2[...], 0.0)
    h = jnp.sum(_bf(h).astype(jnp.float32) * _bf(We3[...]).astype(jnp.float32),
                axis=1, keepdims=True)
    out[...] = jnp.maximum(h + be3[0], 0.0)


def _edge_mlp(edge_attr, We1, be1, We2, be2, We3, be3):
    grid = E // _EB
    full = lambda a: pl.BlockSpec(a.shape, lambda i: (0,) * a.ndim)
    return pl.pallas_call(
        _edge_mlp_body,
        grid=(grid,),
        in_specs=[
            pl.BlockSpec((_EB, 5), lambda i: (i, 0)),
            full(We1), full(be1), full(We2), full(be2), full(We3),
            pl.BlockSpec(memory_space=pltpu.SMEM),
        ],
        out_specs=pl.BlockSpec((_EB, 1), lambda i: (i, 0)),
        out_shape=_f32(E, 1),
    )(edge_attr, We1, be1, We2, be2, We3, be3)


_SC_MESH = plsc.VectorSubcoreMesh(core_axis_name="c", subcore_axis_name="s")
_SC_PARAMS = pltpu.CompilerParams(use_tc_tiling_on_sc=False)


@functools.partial(
    pl.kernel,
    out_type=_f32(NC, N_PAD),
    mesh=_SC_MESH,
    compiler_params=_SC_PARAMS,
    scratch_types=[
        pltpu.VMEM((NCHUNK, CK), jnp.int32),
        pltpu.VMEM((NCHUNK, CK), jnp.float32),
        pltpu.VMEM((RPW,), jnp.float32),
        pltpu.VMEM_SHARED((N_PAD,), jnp.float32),
    ],
)
def _degree(col_hbm, ew_hbm, out_hbm, col_v, ew_v, zb, deg_sh):
    c = lax.axis_index("c")
    s = lax.axis_index("s")
    wid = c * NS + s

    pltpu.sync_copy(col_hbm.at[wid], col_v)
    pltpu.sync_copy(ew_hbm.at[wid], ew_v)

    zero16 = jnp.zeros((16,), jnp.float32)

    @pl.loop(0, RPW // 16)
    def _(i):
        zb[pl.ds(pl.multiple_of(i * 16, 16), 16)] = zero16

    pltpu.sync_copy(zb, deg_sh.at[pl.ds(pl.multiple_of(s * RPW, RPW), RPW)])
    plsc.subcore_barrier()

    @pl.loop(0, NCHUNK)
    def _(j):
        pltpu.sync_copy(ew_v.at[j], deg_sh.at[col_v.at[j]], add=True)

    plsc.subcore_barrier()
    pltpu.sync_copy(deg_sh.at[pl.ds(pl.multiple_of(s * RPW, RPW), RPW)],
                    out_hbm.at[c, pl.ds(pl.multiple_of(s * RPW, RPW), RPW)])


_NB = 1000


def _scale_body(x, Wg, da, db, y, dinv):
    xl = _dot_t(x[...], Wg[...])
    d = da[...] + db[...] + 1.0
    di = lax.rsqrt(d)
    dinv[...] = di
    y[...] = di * xl


def _scale_xw(x, W_gcn, da, db):
    grid = N // _NB
    return pl.pallas_call(
        _scale_body,
        grid=(grid,),
        in_specs=[
            pl.BlockSpec((_NB, F_IN), lambda i: (i, 0)),
            pl.BlockSpec(W_gcn.shape, lambda i: (0, 0)),
            pl.BlockSpec((_NB, 1), lambda i: (i, 0)),
            pl.BlockSpec((_NB, 1), lambda i: (i, 0)),
        ],
        out_specs=[
            pl.BlockSpec((_NB, NHID), lambda i: (i, 0)),
            pl.BlockSpec((_NB, 1), lambda i: (i, 0)),
        ],
        out_shape=[_f32(N, NHID), _f32(N, 1)],
    )(x, W_gcn, da, db)


_ZR = 64


@functools.partial(
    pl.kernel,
    out_type=_f32(NC, N_PAD, NHID),
    mesh=_SC_MESH,
    compiler_params=_SC_PARAMS,
    scratch_types=[
        pltpu.VMEM((NCHUNK, CK), jnp.int32),
        pltpu.VMEM((NCHUNK, CK), jnp.int32),
        pltpu.VMEM((NCHUNK, CK), jnp.float32),
        pltpu.VMEM((NBUF, CK, NHID), jnp.float32),
        pltpu.VMEM((NBUF, CK, NHID), jnp.float32),
        pltpu.VMEM((_ZR, NHID), jnp.float32),
        pltpu.VMEM_SHARED((N_PAD, NHID), jnp.float32),
        pltpu.SemaphoreType.DMA((NBUF,)),
        pltpu.SemaphoreType.DMA((NBUF,)),
    ],
)
def _messages(row_hbm, col_hbm, ew_hbm, y_hbm, out_hbm,
              row_v, col_v, ew_v, rows, sdst, zbuf, S, gsem, ssem):
    c = lax.axis_index("c")
    s = lax.axis_index("s")
    wid = c * NS + s

    pltpu.sync_copy(row_hbm.at[wid], row_v)
    pltpu.sync_copy(col_hbm.at[wid], col_v)
    pltpu.sync_copy(ew_hbm.at[wid], ew_v)

    zero16 = jnp.zeros((16,), jnp.float32)

    @pl.loop(0, _ZR)
    def _(r):
        for q in range(NHID // 16):
            zbuf[r, pl.ds(q * 16, 16)] = zero16

    @pl.loop(0, RPW // _ZR)
    def _(k):
        pltpu.sync_copy(zbuf, S.at[pl.ds(s * RPW + k * _ZR, _ZR)])

    plsc.subcore_barrier()

    def start_gather(j, b):
        pltpu.async_copy(y_hbm.at[row_v.at[j]], rows.at[b], gsem.at[b])

    for b in range(PF):
        start_gather(b, b)

    @pl.loop(0, NCHUNK, step=NBUF)
    def _(j0):
        for b in range(NBUF):
            j = j0 + b
            pltpu.make_async_copy(y_hbm.at[row_v.at[j]], rows.at[b],
                                  gsem.at[b]).wait()

            @pl.when(j >= NBUF)
            def _():
                pltpu.make_async_copy(sdst.at[b], S.at[col_v.at[j - NBUF]],
                                      ssem.at[b]).wait()

            @pl.loop(0, CK // 16)
            def _(g):
                ewg = ew_v[j, pl.ds(g * 16, 16)]
                for ee in range(16):
                    sv = jnp.zeros((16,), jnp.float32) + ewg[ee]
                    e = g * 16 + ee
                    for q in range(NHID // 16):
                        sdst[b, e, pl.ds(q * 16, 16)] = (
                            rows[b, e, pl.ds(q * 16, 16)] * sv)

            pltpu.async_copy(sdst.at[b], S.at[col_v.at[j]], ssem.at[b],
                             add=True)

            @pl.when(j + PF < NCHUNK)
            def _():
                start_gather(j + PF, (b + PF) % NBUF)

    for b in range(NBUF):
        pltpu.make_async_copy(sdst.at[b], S.at[col_v.at[NCHUNK - NBUF + b]],
                              ssem.at[b]).wait()

    plsc.subcore_barrier()
    pltpu.sync_copy(S.at[pl.ds(s * RPW, RPW)], out_hbm.at[c, pl.ds(s * RPW, RPW)])


def _head_body(s0, s1, y, dinv, batch, bg, Wb1, bb1, Wout, bout, out, acc):
    i = pl.program_id(0)

    @pl.when(i == 0)
    def _():
        acc[...] = jnp.zeros_like(acc)

    o = dinv[...] * (s0[...] + s1[...] + y[...]) + bg[...]
    x1 = jnp.maximum(o, 0.0)
    gids = lax.broadcasted_iota(jnp.int32, (_NB, G), 1)
    oh = (batch[...] == gids).astype(jnp.float32)
    acc[...] += lax.dot_general(oh, x1, (((0,), (0,)), ((), ())),
                                preferred_element_type=jnp.float32,
                                precision=lax.Precision.HIGHEST)

    @pl.when(i == pl.num_programs(0) - 1)
    def _():
        hb = jnp.maximum(_dot_t(acc[...], Wb1[...]) + bb1[...], 0.0)
        lg = jnp.sum(_bf(hb).astype(jnp.float32)
                     * _bf(Wout[...]).astype(jnp.float32),
                     axis=1, keepdims=True)
        out[...] = lg + bout[0]


def _head(s0, s1, y, dinv, batch, bg, Wb1, bb1, Wout, bout):
    grid = N // _NB
    full = lambda a: pl.BlockSpec(a.shape, lambda i: (0,) * a.ndim)
    blk = lambda w: pl.BlockSpec((_NB, w), lambda i: (i, 0))
    return pl.pallas_call(
        _head_body,
        grid=(grid,),
        in_specs=[blk(NHID), blk(NHID), blk(NHID), blk(1), blk(1),
                  full(bg), full(Wb1), full(bb1), full(Wout),
                  pl.BlockSpec(memory_space=pltpu.SMEM)],
        out_specs=pl.BlockSpec((G, 1), lambda i: (0, 0)),
        out_shape=_f32(G, 1),
        scratch_shapes=[pltpu.VMEM((G, NHID), jnp.float32)],
    )(s0, s1, y, dinv, batch, bg, Wb1, bb1, Wout, bout)


def kernel(x, edge_index, edge_attr, batch, W_gcn, b_gcn, We1, be1, We2, be2,
           We3, be3, Wb1, bb1, Wout, bout):
    row = edge_index[0]
    col = edge_index[1]

    ew2 = _edge_mlp(edge_attr, We1, be1.reshape(1, NHID), We2,
                    be2.reshape(1, NHID), We3, be3)
    ew = ew2.reshape(E)

    pad = ((0, 0), (0, EPW_P - EPW))
    row_c = jnp.pad(row.reshape(NW, EPW), pad).reshape(NW, NCHUNK, CK)
    col_c = jnp.pad(col.reshape(NW, EPW), pad).reshape(NW, NCHUNK, CK)
    ew_c = jnp.pad(ew.reshape(NW, EPW), pad).reshape(NW, NCHUNK, CK)

    deg2 = _degree(col_c, ew_c)
    da = deg2[0, :N].reshape(N, 1)
    db = deg2[1, :N].reshape(N, 1)

    y, dinv = _scale_xw(x, W_gcn, da, db)

    s2 = _messages(row_c, col_c, ew_c, y)
    s0 = s2[0, :N]
    s1 = s2[1, :N]

    return _head(s0, s1, y, dinv, batch.reshape(N, 1), b_gcn.reshape(1, NHID),
                 Wb1, bb1.reshape(1, NHID), Wout, bout)
